# baseline (speedup 1.0000x reference)
import functools

import jax
import jax.numpy as jnp
from jax import lax
from jax.experimental import pallas as pl
from jax.experimental.pallas import tpu as pltpu
from jax.experimental.pallas import tpu_sc as plsc

N = 1000000
N_D = 32
BATCH = 16384

_info = plsc.get_sparse_core_info()
_NC, _NS = _info.num_cores, _info.num_subcores
_NW = _NC * _NS
_BPW = BATCH // _NW
_TOT = N_D * _BPW


def _gather_body(idx_hbm, wa_flat, ws_flat, oa_hbm, os_hbm,
                 idx_v, exp_v, ga_v, gs_v, sem_i, sem_a, sem_s):
    wid = lax.axis_index("s") * _NC + lax.axis_index("c")
    base = wid * _BPW
    pltpu.async_copy(idx_hbm.at[pl.ds(base, _BPW)], idx_v, sem_i).wait()

    def fill(b, carry):
        j0 = b * 16
        iv = idx_v[pl.ds(j0, 16)]
        for d in range(N_D):
            exp_v[pl.ds(d * _BPW + j0, 16)] = iv + d * N
        return carry

    lax.fori_loop(0, _BPW // 16, fill, 0)

    ca = pltpu.async_copy(wa_flat.at[exp_v], ga_v, sem_a)
    cs = pltpu.async_copy(ws_flat.at[exp_v], gs_v, sem_s)
    ca.wait()
    for d in range(N_D):
        pltpu.async_copy(ga_v.at[pl.ds(d * _BPW, _BPW)],
                         oa_hbm.at[d, pl.ds(base, _BPW)], sem_a)
    cs.wait()
    for d in range(N_D):
        pltpu.async_copy(gs_v.at[pl.ds(d * _BPW, _BPW)],
                         os_hbm.at[d, pl.ds(base, _BPW)], sem_s)
    pltpu.make_async_copy(ga_v, oa_hbm.at[0, pl.ds(0, _TOT)], sem_a).wait()
    pltpu.make_async_copy(gs_v, os_hbm.at[0, pl.ds(0, _TOT)], sem_s).wait()


@jax.jit
def kernel(idx, Wa, Ws):
    mesh = plsc.VectorSubcoreMesh(core_axis_name="c", subcore_axis_name="s")
    run = functools.partial(
        pl.kernel,
        mesh=mesh,
        out_type=(
            jax.ShapeDtypeStruct((N_D, BATCH), jnp.float32),
            jax.ShapeDtypeStruct((N_D, BATCH), jnp.float32),
        ),
        scratch_types=[
            pltpu.VMEM((_BPW,), jnp.int32),
            pltpu.VMEM((_TOT,), jnp.int32),
            pltpu.VMEM((_TOT,), jnp.float32),
            pltpu.VMEM((_TOT,), jnp.float32),
            pltpu.SemaphoreType.DMA,
            pltpu.SemaphoreType.DMA,
            pltpu.SemaphoreType.DMA,
        ],
        compiler_params=pltpu.CompilerParams(use_tc_tiling_on_sc=False),
    )(_gather_body)
    wa_flat = jnp.concatenate([Wa[:, d] for d in range(N_D)])
    ws_flat = jnp.concatenate([Ws[:, d] for d in range(N_D)])
    za_t, zs_t = run(idx, wa_flat, ws_flat)
    return (za_t.T, zs_t.T)


# speedup vs baseline: 4.2786x; 4.2786x over previous
import functools

import jax
import jax.numpy as jnp
from jax import lax
from jax.experimental import pallas as pl
from jax.experimental.pallas import tpu as pltpu
from jax.experimental.pallas import tpu_sc as plsc

N = 1000000
N_D = 32
BATCH = 16384

_info = plsc.get_sparse_core_info()
_NC, _NS = _info.num_cores, _info.num_subcores
_NW = _NC * _NS
_BPW = BATCH // _NW


def _gather_body(idx_hbm, tab_hbm, out_hbm, idx_v, rows_v, sem_i, sem_g):
    wid = lax.axis_index("s") * _NC + lax.axis_index("c")
    base = wid * _BPW
    pltpu.async_copy(idx_hbm.at[pl.ds(base, _BPW)], idx_v, sem_i).wait()
    pltpu.async_copy(tab_hbm.at[idx_v], rows_v, sem_g).wait()
    pltpu.async_copy(rows_v, out_hbm.at[pl.ds(base, _BPW), :], sem_g).wait()


@jax.jit
def kernel(idx, Wa, Ws):
    mesh = plsc.VectorSubcoreMesh(core_axis_name="c", subcore_axis_name="s")
    run = functools.partial(
        pl.kernel,
        mesh=mesh,
        out_type=jax.ShapeDtypeStruct((BATCH, 128), jnp.float32),
        scratch_types=[
            pltpu.VMEM((_BPW,), jnp.int32),
            pltpu.VMEM((_BPW, 128), jnp.float32),
            pltpu.SemaphoreType.DMA,
            pltpu.SemaphoreType.DMA,
        ],
    )(_gather_body)
    tab = jnp.concatenate(
        [Wa, Ws, jnp.zeros((N, 128 - 2 * N_D), jnp.float32)], axis=1)
    out = run(idx, tab)
    return (out[:, :N_D], out[:, N_D:2 * N_D])


# speedup vs baseline: 4.2842x; 1.0013x over previous
import functools

import jax
import jax.numpy as jnp
from jax import lax
from jax.experimental import pallas as pl
from jax.experimental.pallas import tpu as pltpu
from jax.experimental.pallas import tpu_sc as plsc

N = 1000000
N_D = 32
BATCH = 16384

_info = plsc.get_sparse_core_info()
_NC, _NS = _info.num_cores, _info.num_subcores
_NW = _NC * _NS
_BPW = BATCH // _NW


def _gather_body(idx_hbm, tab_hbm, out_hbm, idx_v, rows_v, sem_i, sem_g):
    wid = lax.axis_index("s") * _NC + lax.axis_index("c")
    base = wid * _BPW
    pltpu.async_copy(idx_hbm.at[pl.ds(base, _BPW)], idx_v, sem_i).wait()
    pltpu.async_copy(tab_hbm.at[idx_v], rows_v, sem_g).wait()
    pltpu.async_copy(rows_v, out_hbm.at[pl.ds(base, _BPW), :], sem_g).wait()


@jax.jit
def kernel(idx, Wa, Ws):
    mesh = plsc.VectorSubcoreMesh(core_axis_name="c", subcore_axis_name="s")
    run = functools.partial(
        pl.kernel,
        mesh=mesh,
        out_type=jax.ShapeDtypeStruct((BATCH, 128), jnp.float32),
        scratch_types=[
            pltpu.VMEM((_BPW,), jnp.int32),
            pltpu.VMEM((_BPW, 128), jnp.float32),
            pltpu.SemaphoreType.DMA,
            pltpu.SemaphoreType.DMA,
        ],
    )(_gather_body)
    tab = jnp.pad(jnp.concatenate([Wa, Ws], axis=1),
                  ((0, 0), (0, 128 - 2 * N_D)))
    out = run(idx, tab)
    return (out[:, :N_D], out[:, N_D:2 * N_D])
